# baseline (speedup 1.0000x reference)
import functools

import jax
import jax.numpy as jnp
from jax import lax
from jax.experimental import pallas as pl
from jax.experimental.pallas import tpu as pltpu
from jax.experimental.pallas import tpu_sc as plsc

_K = 10
_NCLS = 256


def _knn_body(n, k, xb_ref, xf_ref, xsq_ref, idx_ref, xd_ref):
    xb = xb_ref[...]
    blk = xb.shape[0]
    gx = lax.dot_general(xb, xf_ref[...], (((1,), (1,)), ((), ())),
                         preferred_element_type=jnp.float32)
    xsq_b = jnp.sum(xb * xb, axis=1, keepdims=True)
    scores = xsq_b + xsq_ref[...] - 2.0 * gx

    iota = lax.broadcasted_iota(jnp.int32, (blk, n), 1)
    mask13 = jnp.int32(0x1FFF)
    keysf = lax.bitcast_convert_type(
        (lax.bitcast_convert_type(scores, jnp.int32) & ~mask13) | iota,
        jnp.float32)

    ncls = _NCLS
    inf = jnp.float32(jnp.inf)
    m1 = keysf[:, 0:ncls]
    m2 = jnp.full((blk, ncls), inf)
    m3 = m2
    for j in range(1, n // ncls):
        x = keysf[:, j * ncls:(j + 1) * ncls]
        t1 = jnp.minimum(m1, x)
        l1 = jnp.maximum(m1, x)
        t2 = jnp.minimum(m2, l1)
        l2 = jnp.maximum(m2, l1)
        m3 = jnp.minimum(m3, l2)
        m1 = t1
        m2 = t2

    prev = None
    for t in range(k + 1):
        if t == 0:
            m = jnp.min(m1, axis=1, keepdims=True)
        else:
            c1 = jnp.min(jnp.where(m1 > prev, m1, inf), axis=1,
                         keepdims=True)
            c2 = jnp.min(jnp.where(m2 > prev, m2, inf), axis=1,
                         keepdims=True)
            c3 = jnp.min(jnp.where(m3 > prev, m3, inf), axis=1,
                         keepdims=True)
            m = jnp.minimum(jnp.minimum(c1, c2), c3)
            mi = lax.bitcast_convert_type(m, jnp.int32)
            idx_ref[:, t - 1:t] = mi & mask13
            xval = lax.bitcast_convert_type(mi & ~mask13, jnp.float32)
            xd_ref[:, t - 1:t] = jnp.sqrt(jnp.maximum(xval, 0.0))
        prev = m


def _sc_gather_rows(fidx, table):
    nrows = fidx.shape[0]
    d = table.shape[1]
    info = plsc.get_sparse_core_info()
    nw = info.num_cores * info.num_subcores
    b_per_w = nrows // nw
    chunk = b_per_w
    for c in (512, 256, 128, 64, 32, 16, 8):
        if b_per_w % c == 0:
            chunk = c
            break
    nchunks = b_per_w // chunk
    mesh = plsc.VectorSubcoreMesh(core_axis_name="c", subcore_axis_name="s")

    @functools.partial(
        pl.kernel, mesh=mesh,
        out_type=jax.ShapeDtypeStruct((nrows, d), jnp.float32),
        scratch_types=[
            pltpu.VMEM((chunk,), jnp.int32),
            pltpu.VMEM((chunk, d), jnp.float32),
            pltpu.SemaphoreType.DMA,
        ],
    )
    def gather_rows(idx_hbm, table_hbm, out_hbm, idx_v, rows_v, sem):
        wid = lax.axis_index("s") * info.num_cores + lax.axis_index("c")
        base = wid * b_per_w
        for cch in range(nchunks):
            off = base + cch * chunk
            pltpu.sync_copy(idx_hbm.at[pl.ds(off, chunk)], idx_v)
            pltpu.async_copy(table_hbm.at[idx_v], rows_v, sem).wait()
            pltpu.sync_copy(rows_v, out_hbm.at[pl.ds(off, chunk)])

    return gather_rows(fidx, table)


def _loss_body(n, k, nblocks, zn_ref, zb_ref, xd_ref, out_ref):
    i = pl.program_id(0)
    zb = zb_ref[...]
    rb, dz = zb.shape
    xdb = xd_ref[...]
    ones = jnp.ones((dz, 128), jnp.float32)
    zds = []
    xds = []
    for t in range(k):
        d = zn_ref[t] - zb
        s = lax.dot_general(d * d, ones, (((1,), (0,)), ((), ())),
                            preferred_element_type=jnp.float32)[:, :1]
        zds.append(jnp.sqrt(jnp.maximum(s, 0.0)))
        xds.append(xdb[:, t:t + 1])
    zmax = jnp.clip(functools.reduce(jnp.maximum, zds), 1e-8, None)
    xmax = jnp.clip(functools.reduce(jnp.maximum, xds), 1e-8, None)
    contrib = sum(jnp.abs(zd / zmax - xd / xmax) for xd, zd in zip(xds, zds))
    total = jnp.reshape(jnp.sum(contrib), (1, 1))

    @pl.when(i == 0)
    def _init():
        out_ref[...] = jnp.zeros((1, 1), jnp.float32)

    out_ref[...] += total


def _knn_half(Xh, X, xsq, n, blk):
    nh = Xh.shape[0]
    dx = X.shape[1]
    return pl.pallas_call(
        functools.partial(_knn_body, n, _K),
        grid=(nh // blk,),
        in_specs=[
            pl.BlockSpec((blk, dx), lambda i: (i, 0)),
            pl.BlockSpec((n, dx), lambda i: (0, 0)),
            pl.BlockSpec((1, n), lambda i: (0, 0)),
        ],
        out_specs=[
            pl.BlockSpec((blk, 16), lambda i: (i, 0)),
            pl.BlockSpec((blk, 16), lambda i: (i, 0)),
        ],
        out_shape=[
            jax.ShapeDtypeStruct((nh, 16), jnp.int32),
            jax.ShapeDtypeStruct((nh, 16), jnp.float32),
        ],
        compiler_params=pltpu.CompilerParams(
            dimension_semantics=("arbitrary",)),
    )(Xh, X, xsq)


def _loss_half(zn, zh, xd, n, rb):
    nh = zh.shape[0]
    dz = zh.shape[1]
    nlb = nh // rb
    return pl.pallas_call(
        functools.partial(_loss_body, n, _K, nlb),
        grid=(nlb,),
        in_specs=[
            pl.BlockSpec((_K, rb, dz), lambda i: (0, i, 0)),
            pl.BlockSpec((rb, dz), lambda i: (i, 0)),
            pl.BlockSpec((rb, 16), lambda i: (i, 0)),
        ],
        out_specs=pl.BlockSpec((1, 1), lambda i: (0, 0)),
        out_shape=jax.ShapeDtypeStruct((1, 1), jnp.float32),
        compiler_params=pltpu.CompilerParams(
            dimension_semantics=("arbitrary",)),
    )(zn, zh, xd)


def kernel(z, X):
    n, dx = X.shape
    dz = z.shape[1]
    blk = 128 if n % 128 == 0 else n
    xsq = jnp.sum(X * X, axis=1)[None, :]

    nhalves = 2 if n % (2 * 1024) == 0 else 1
    nh = n // nhalves
    rb = 1024 if nh % 1024 == 0 else nh
    sums = []
    for h in range(nhalves):
        Xh = X[h * nh:(h + 1) * nh]
        idx, xd = _knn_half(Xh, X, xsq, n, blk)
        fidx = idx[:, :_K].T.reshape(-1)
        zn = _sc_gather_rows(fidx, z).reshape(_K, nh, dz)
        sums.append(_loss_half(zn, z[h * nh:(h + 1) * nh], xd, n, rb))
    return sum(s[0, 0] for s in sums) / jnp.float32(n * _K)


# speedup vs baseline: 1.4413x; 1.4413x over previous
import functools

import jax
import jax.numpy as jnp
from jax import lax
from jax.experimental import pallas as pl
from jax.experimental.pallas import tpu as pltpu
from jax.experimental.pallas import tpu_sc as plsc

_K = 10
_NCLS = 256


def _knn_body(n, k, xb_ref, xf_ref, xsq_ref, pk_ref):
    xb = xb_ref[...]
    blk = xb.shape[0]
    gx = lax.dot_general(xb, xf_ref[...], (((1,), (1,)), ((), ())),
                         preferred_element_type=jnp.float32)
    xsq_b = jnp.sum(xb * xb, axis=1, keepdims=True)
    scores = xsq_b + xsq_ref[...] - 2.0 * gx

    iota = lax.broadcasted_iota(jnp.int32, (blk, n), 1)
    mask13 = jnp.int32(0x1FFF)
    keysf = lax.bitcast_convert_type(
        (lax.bitcast_convert_type(scores, jnp.int32) & ~mask13) | iota,
        jnp.float32)

    ncls = _NCLS
    inf = jnp.float32(jnp.inf)
    m1 = keysf[:, 0:ncls]
    m2 = jnp.full((blk, ncls), inf)
    m3 = m2
    for j in range(1, n // ncls):
        x = keysf[:, j * ncls:(j + 1) * ncls]
        t1 = jnp.minimum(m1, x)
        l1 = jnp.maximum(m1, x)
        t2 = jnp.minimum(m2, l1)
        l2 = jnp.maximum(m2, l1)
        m3 = jnp.minimum(m3, l2)
        m1 = t1
        m2 = t2

    prev = None
    for t in range(k + 1):
        if t == 0:
            m = jnp.min(m1, axis=1, keepdims=True)
        else:
            cand = jnp.minimum(
                jnp.minimum(jnp.where(m1 > prev, m1, inf),
                            jnp.where(m2 > prev, m2, inf)),
                jnp.where(m3 > prev, m3, inf))
            m = jnp.min(cand, axis=1, keepdims=True)
            pk_ref[:, t - 1:t] = m
        prev = m


def _sc_gather_rows(fidx, table):
    nrows = fidx.shape[0]
    d = table.shape[1]
    info = plsc.get_sparse_core_info()
    nw = info.num_cores * info.num_subcores
    b_per_w = nrows // nw
    chunk = b_per_w
    for c in (512, 256, 128, 64, 32, 16, 8):
        if b_per_w % c == 0:
            chunk = c
            break
    nchunks = b_per_w // chunk
    mesh = plsc.VectorSubcoreMesh(core_axis_name="c", subcore_axis_name="s")

    @functools.partial(
        pl.kernel, mesh=mesh,
        out_type=jax.ShapeDtypeStruct((nrows, d), jnp.float32),
        scratch_types=[
            pltpu.VMEM((chunk,), jnp.int32),
            pltpu.VMEM((chunk, d), jnp.float32),
            pltpu.SemaphoreType.DMA,
        ],
    )
    def gather_rows(idx_hbm, table_hbm, out_hbm, idx_v, rows_v, sem):
        wid = lax.axis_index("s") * info.num_cores + lax.axis_index("c")
        base = wid * b_per_w
        for cch in range(nchunks):
            off = base + cch * chunk
            pltpu.sync_copy(idx_hbm.at[pl.ds(off, chunk)], idx_v)
            pltpu.async_copy(table_hbm.at[idx_v], rows_v, sem).wait()
            pltpu.sync_copy(rows_v, out_hbm.at[pl.ds(off, chunk)])

    return gather_rows(fidx, table)


def _loss_body(n, k, nblocks, zn_ref, zb_ref, pk_ref, out_ref):
    i = pl.program_id(0)
    zb = zb_ref[...]
    rb, dz = zb.shape
    pkb = pk_ref[...]
    mask13 = jnp.int32(0x1FFF)
    ones = jnp.ones((dz, 128), jnp.float32)
    zds = []
    xds = []
    for t in range(k):
        d = zn_ref[t] - zb
        s = lax.dot_general(d * d, ones, (((1,), (0,)), ((), ())),
                            preferred_element_type=jnp.float32)[:, :1]
        zds.append(jnp.sqrt(jnp.maximum(s, 0.0)))
        xbits = lax.bitcast_convert_type(pkb[:, t:t + 1], jnp.int32)
        xval = lax.bitcast_convert_type(xbits & ~mask13, jnp.float32)
        xds.append(jnp.sqrt(jnp.maximum(xval, 0.0)))
    zmax = jnp.clip(functools.reduce(jnp.maximum, zds), 1e-8, None)
    xmax = jnp.clip(functools.reduce(jnp.maximum, xds), 1e-8, None)
    contrib = sum(jnp.abs(zd / zmax - xd / xmax) for xd, zd in zip(xds, zds))
    total = jnp.reshape(jnp.sum(contrib), (1, 1))

    @pl.when(i == 0)
    def _init():
        out_ref[...] = jnp.zeros((1, 1), jnp.float32)

    out_ref[...] += total


def _knn_half(Xh, X, xsq, n, blk):
    nh = Xh.shape[0]
    dx = X.shape[1]
    return pl.pallas_call(
        functools.partial(_knn_body, n, _K),
        grid=(nh // blk,),
        in_specs=[
            pl.BlockSpec((blk, dx), lambda i: (i, 0)),
            pl.BlockSpec((n, dx), lambda i: (0, 0)),
            pl.BlockSpec((1, n), lambda i: (0, 0)),
        ],
        out_specs=pl.BlockSpec((blk, 16), lambda i: (i, 0)),
        out_shape=jax.ShapeDtypeStruct((nh, 16), jnp.float32),
        compiler_params=pltpu.CompilerParams(
            dimension_semantics=("arbitrary",)),
    )(Xh, X, xsq)


def _loss_half(zn, zh, pk, n, rb):
    nh = zh.shape[0]
    dz = zh.shape[1]
    nlb = nh // rb
    return pl.pallas_call(
        functools.partial(_loss_body, n, _K, nlb),
        grid=(nlb,),
        in_specs=[
            pl.BlockSpec((_K, rb, dz), lambda i: (0, i, 0)),
            pl.BlockSpec((rb, dz), lambda i: (i, 0)),
            pl.BlockSpec((rb, 16), lambda i: (i, 0)),
        ],
        out_specs=pl.BlockSpec((1, 1), lambda i: (0, 0)),
        out_shape=jax.ShapeDtypeStruct((1, 1), jnp.float32),
        compiler_params=pltpu.CompilerParams(
            dimension_semantics=("arbitrary",)),
    )(zn, zh, pk)


def kernel(z, X):
    n, dx = X.shape
    dz = z.shape[1]
    blk = 256 if n % 256 == 0 else n
    xsq = jnp.sum(X * X, axis=1)[None, :]

    nhalves = 2 if n % (2 * 1024) == 0 else 1
    nh = n // nhalves
    rb = 1024 if nh % 1024 == 0 else nh
    sums = []
    for h in range(nhalves):
        Xh = X[h * nh:(h + 1) * nh]
        pk = _knn_half(Xh, X, xsq, n, blk)
        fidx = (lax.bitcast_convert_type(pk[:, :_K], jnp.int32)
                & 0x1FFF).T.reshape(-1)
        zn = _sc_gather_rows(fidx, z).reshape(_K, nh, dz)
        sums.append(_loss_half(zn, z[h * nh:(h + 1) * nh], pk, n, rb))
    return sum(s[0, 0] for s in sums) / jnp.float32(n * _K)
